# Initial kernel scaffold; baseline (speedup 1.0000x reference)
import jax
import jax.numpy as jnp
from jax.experimental import pallas as pl


def kernel(queries, keys):
    raise NotImplementedError("write your pallas kernel here")


# speedup vs baseline: 2.0218x; 2.0218x over previous
import functools

import jax
import jax.numpy as jnp
from jax.experimental import pallas as pl
from jax.experimental.pallas import tpu as pltpu
from jax.experimental.pallas import tpu_sc as plsc

B = 1024
QD = 1024
D = 768
KTOT = 100000
KB = 2048
NK = 49
KPAD = NK * KB
TOPK = 5
NEG = -1e30

W = NK * 8
WPAD = 400
NCH = WPAD // 16

NW = 32
QPW = B // NW


def _phase1_body(q_ref, keys_ref, vals_ref, idx_ref, x_s):
    k = pl.program_id(0)

    @pl.when(k == 0)
    def _():
        q = q_ref[...]
        mean = jnp.mean(q, axis=1, keepdims=True)
        var = jnp.mean((q - mean) ** 2, axis=1, keepdims=True)
        xn = (q - mean) / jnp.sqrt(var + 1e-5)
        xt = xn[:, :D]
        nrm = jnp.sqrt(jnp.sum(xt * xt, axis=1, keepdims=True))
        x_s[...] = xt / jnp.maximum(nrm, 1e-12)

    x = x_s[...]
    kb = keys_ref[...]
    sims = jax.lax.dot_general(
        x, kb, (((1,), (1,)), ((), ())), preferred_element_type=jnp.float32)
    colf = (jax.lax.broadcasted_iota(jnp.int32, (B, KB), 1).astype(jnp.float32)
            + jnp.float32(k * KB))
    s = jnp.where(colf < jnp.float32(KTOT), sims, NEG)
    vs, idfs = [], []
    m = jnp.max(s, axis=1, keepdims=True)
    for t in range(TOPK):
        candf = jnp.where(s == m, colf, jnp.float32(3e38))
        amf = jnp.min(candf, axis=1, keepdims=True)
        vs.append(m)
        idfs.append(amf)
        if t < TOPK - 1:
            m = jnp.max(jnp.where(s < m, s, NEG), axis=1, keepdims=True)
    vs.append(jnp.full((B, 3), NEG, jnp.float32))
    idfs.append(jnp.zeros((B, 3), jnp.float32))
    vals_ref[0] = jnp.concatenate(vs, axis=1)
    idx_ref[0] = jnp.concatenate(idfs, axis=1).astype(jnp.int32)


_phase1 = pl.pallas_call(
    _phase1_body,
    grid=(NK,),
    in_specs=[
        pl.BlockSpec((B, QD), lambda k: (0, 0)),
        pl.BlockSpec((KB, D), lambda k: (k, 0)),
    ],
    out_specs=[
        pl.BlockSpec((1, B, 8), lambda k: (k, 0, 0)),
        pl.BlockSpec((1, B, 8), lambda k: (k, 0, 0)),
    ],
    out_shape=[
        jax.ShapeDtypeStruct((NK, B, 8), jnp.float32),
        jax.ShapeDtypeStruct((NK, B, 8), jnp.int32),
    ],
    scratch_shapes=[pltpu.VMEM((B, D), jnp.float32)],
)


def _smax(v):
    return jax.lax.sort(v)[15]


def _smin(v):
    return jax.lax.sort(v)[0]


def _sc_merge_body(cv_hbm, ci_hbm, ov_hbm, oi_hbm, v_s, i_s, obv, obi):
    c = jax.lax.axis_index("c")
    s = jax.lax.axis_index("s")
    wid = s * 2 + c
    base = wid * QPW
    pltpu.sync_copy(cv_hbm.at[pl.ds(base, QPW)], v_s)
    pltpu.sync_copy(ci_hbm.at[pl.ds(base, QPW)], i_s)
    lanes = jax.lax.iota(jnp.int32, 16)

    def per_query(j, carry):
        vacc = jnp.zeros((16,), jnp.float32)
        iacc = jnp.zeros((16,), jnp.int32)
        for t in range(TOPK):
            def per_chunk(cidx, bc):
                best, bpos = bc
                v = v_s[j, pl.ds(cidx * 16, 16)]
                m = v > best
                pos = cidx * 16 + lanes
                return jnp.where(m, v, best), jnp.where(m, pos, bpos)

            best, bpos = jax.lax.fori_loop(
                0, NCH, per_chunk,
                (jnp.full((16,), NEG, jnp.float32),
                 jnp.full((16,), 2 ** 30, jnp.int32)))
            gmax = _smax(best)
            cand = jnp.where(best == gmax, bpos, jnp.int32(2 ** 30))
            pmin = _smin(cand)
            ch = pmin // 16
            ln = pmin - ch * 16
            vv = v_s[j, pl.ds(ch * 16, 16)]
            iv = i_s[j, pl.ds(ch * 16, 16)]
            idxv = _smin(jnp.where(lanes == ln, iv, jnp.int32(2 ** 30)))
            v_s[j, pl.ds(ch * 16, 16)] = jnp.where(lanes == ln, NEG, vv)
            vacc = jnp.where(lanes == t, gmax, vacc)
            iacc = jnp.where(lanes == t, idxv, iacc)
        obv[j, pl.ds(0, 16)] = vacc
        obi[j, pl.ds(0, 16)] = iacc
        return carry

    jax.lax.fori_loop(0, QPW, per_query, 0)
    pltpu.sync_copy(obv, ov_hbm.at[pl.ds(base, QPW)])
    pltpu.sync_copy(obi, oi_hbm.at[pl.ds(base, QPW)])


@functools.cache
def _build_sc_merge():
    return functools.partial(
        pl.kernel,
        out_type=[
            jax.ShapeDtypeStruct((B, 16), jnp.float32),
            jax.ShapeDtypeStruct((B, 16), jnp.int32),
        ],
        mesh=plsc.VectorSubcoreMesh(core_axis_name="c", subcore_axis_name="s"),
        compiler_params=pltpu.CompilerParams(needs_layout_passes=False),
        scratch_types=[
            pltpu.VMEM((QPW, WPAD), jnp.float32),
            pltpu.VMEM((QPW, WPAD), jnp.int32),
            pltpu.VMEM((QPW, 16), jnp.float32),
            pltpu.VMEM((QPW, 16), jnp.int32),
        ],
    )(_sc_merge_body)


def kernel(queries, keys):
    keys_p = jnp.pad(keys, ((0, KPAD - KTOT), (0, 0)))
    cv, ci = _phase1(queries, keys_p)
    cv = jnp.transpose(cv, (1, 0, 2)).reshape(B, W)
    ci = jnp.transpose(ci, (1, 0, 2)).reshape(B, W)
    cv = jnp.pad(cv, ((0, 0), (0, WPAD - W)), constant_values=NEG)
    ci = jnp.pad(ci, ((0, 0), (0, WPAD - W)))
    ov, oi = _build_sc_merge()(cv, ci)
    return ov[:, :TOPK], oi[:, :TOPK]


# speedup vs baseline: 2.5492x; 1.2609x over previous
import functools

import jax
import jax.numpy as jnp
from jax.experimental import pallas as pl
from jax.experimental.pallas import tpu as pltpu
from jax.experimental.pallas import tpu_sc as plsc

B = 1024
QD = 1024
D = 768
KTOT = 100000
KB = 2048
NK = 49
KPAD = NK * KB
TOPK = 5
NEG = -1e30

W = NK * 8
WPAD = 400
NCH = WPAD // 16

NW = 32
QPW = B // NW


def _phase1_body(q_ref, keys_ref, vals_ref, idx_ref, x_s, colf_s):
    k = pl.program_id(0)

    @pl.when(k == 0)
    def _():
        q = q_ref[...]
        mean = jnp.mean(q, axis=1, keepdims=True)
        var = jnp.mean((q - mean) ** 2, axis=1, keepdims=True)
        xn = (q - mean) / jnp.sqrt(var + 1e-5)
        xt = xn[:, :D]
        nrm = jnp.sqrt(jnp.sum(xt * xt, axis=1, keepdims=True))
        x_s[...] = xt / jnp.maximum(nrm, 1e-12)
        colf_s[...] = jax.lax.broadcasted_iota(
            jnp.int32, (B, KB), 1).astype(jnp.float32)

    x = x_s[...]
    kb = keys_ref[...]
    sims = jax.lax.dot_general(
        x, kb, (((1,), (1,)), ((), ())), preferred_element_type=jnp.float32)
    colf = colf_s[...]
    thr = jnp.float32(KTOT) - jnp.float32(KB) * k.astype(jnp.float32)
    s = jnp.where(colf < thr, sims, NEG)
    vs, idfs = [], []
    m = jnp.max(s, axis=1, keepdims=True)
    for t in range(TOPK):
        candf = jnp.where(s == m, colf, jnp.float32(3e38))
        amf = jnp.min(candf, axis=1, keepdims=True)
        vs.append(m)
        idfs.append(amf)
        if t < TOPK - 1:
            m = jnp.max(jnp.where(s < m, s, NEG), axis=1, keepdims=True)
    base = jnp.float32(KB) * k.astype(jnp.float32)
    vs.append(jnp.full((B, 3), NEG, jnp.float32))
    idfs.append(jnp.full((B, 3), -base, jnp.float32))
    vals_ref[0] = jnp.concatenate(vs, axis=1)
    idx_ref[0] = (jnp.concatenate(idfs, axis=1) + base).astype(jnp.int32)


_phase1 = pl.pallas_call(
    _phase1_body,
    grid=(NK,),
    in_specs=[
        pl.BlockSpec((B, QD), lambda k: (0, 0)),
        pl.BlockSpec((KB, D), lambda k: (k, 0)),
    ],
    out_specs=[
        pl.BlockSpec((1, B, 8), lambda k: (k, 0, 0)),
        pl.BlockSpec((1, B, 8), lambda k: (k, 0, 0)),
    ],
    out_shape=[
        jax.ShapeDtypeStruct((NK, B, 8), jnp.float32),
        jax.ShapeDtypeStruct((NK, B, 8), jnp.int32),
    ],
    scratch_shapes=[pltpu.VMEM((B, D), jnp.float32),
                    pltpu.VMEM((B, KB), jnp.float32)],
)


def _smax(v):
    return jax.lax.sort(v)[15]


def _smin(v):
    return jax.lax.sort(v)[0]


def _sc_merge_body(cv_hbm, ci_hbm, ov_hbm, oi_hbm, v_s, i_s, obv, obi):
    c = jax.lax.axis_index("c")
    s = jax.lax.axis_index("s")
    wid = s * 2 + c
    base = wid * QPW
    pltpu.sync_copy(cv_hbm.at[pl.ds(base, QPW)], v_s)
    pltpu.sync_copy(ci_hbm.at[pl.ds(base, QPW)], i_s)
    lanes = jax.lax.iota(jnp.int32, 16)

    def per_query(j, carry):
        vacc = jnp.zeros((16,), jnp.float32)
        iacc = jnp.zeros((16,), jnp.int32)
        for t in range(TOPK):
            def per_chunk(cidx, bc):
                best, bpos = bc
                v = v_s[j, pl.ds(cidx * 16, 16)]
                m = v > best
                pos = cidx * 16 + lanes
                return jnp.where(m, v, best), jnp.where(m, pos, bpos)

            best, bpos = jax.lax.fori_loop(
                0, NCH, per_chunk,
                (jnp.full((16,), NEG, jnp.float32),
                 jnp.full((16,), 2 ** 30, jnp.int32)))
            gmax = _smax(best)
            cand = jnp.where(best == gmax, bpos, jnp.int32(2 ** 30))
            pmin = _smin(cand)
            ch = pmin // 16
            ln = pmin - ch * 16
            vv = v_s[j, pl.ds(ch * 16, 16)]
            iv = i_s[j, pl.ds(ch * 16, 16)]
            idxv = _smin(jnp.where(lanes == ln, iv, jnp.int32(2 ** 30)))
            v_s[j, pl.ds(ch * 16, 16)] = jnp.where(lanes == ln, NEG, vv)
            vacc = jnp.where(lanes == t, gmax, vacc)
            iacc = jnp.where(lanes == t, idxv, iacc)
        obv[j, pl.ds(0, 16)] = vacc
        obi[j, pl.ds(0, 16)] = iacc
        return carry

    jax.lax.fori_loop(0, QPW, per_query, 0)
    pltpu.sync_copy(obv, ov_hbm.at[pl.ds(base, QPW)])
    pltpu.sync_copy(obi, oi_hbm.at[pl.ds(base, QPW)])


@functools.cache
def _build_sc_merge():
    return functools.partial(
        pl.kernel,
        out_type=[
            jax.ShapeDtypeStruct((B, 16), jnp.float32),
            jax.ShapeDtypeStruct((B, 16), jnp.int32),
        ],
        mesh=plsc.VectorSubcoreMesh(core_axis_name="c", subcore_axis_name="s"),
        compiler_params=pltpu.CompilerParams(needs_layout_passes=False),
        scratch_types=[
            pltpu.VMEM((QPW, WPAD), jnp.float32),
            pltpu.VMEM((QPW, WPAD), jnp.int32),
            pltpu.VMEM((QPW, 16), jnp.float32),
            pltpu.VMEM((QPW, 16), jnp.int32),
        ],
    )(_sc_merge_body)


def kernel(queries, keys):
    cv, ci = _phase1(queries, keys)
    cv = jnp.transpose(cv, (1, 0, 2)).reshape(B, W)
    ci = jnp.transpose(ci, (1, 0, 2)).reshape(B, W)
    cv = jnp.pad(cv, ((0, 0), (0, WPAD - W)), constant_values=NEG)
    ci = jnp.pad(ci, ((0, 0), (0, WPAD - W)))
    ov, oi = _build_sc_merge()(cv, ci)
    return ov[:, :TOPK], oi[:, :TOPK]


# speedup vs baseline: 2.8513x; 1.1185x over previous
import functools

import jax
import jax.numpy as jnp
from jax.experimental import pallas as pl
from jax.experimental.pallas import tpu as pltpu
from jax.experimental.pallas import tpu_sc as plsc

B = 1024
QD = 1024
D = 768
KTOT = 100000
KB = 1000
NK = 100
UN = 2
NSTEP = NK // UN + 1
TOPK = 5
NEG = -1e30

W = NK * 8
WPAD = W
NCH = WPAD // 16

NW = 32
QPW = B // NW


def _select_top5(s, colf, base):
    vs, idfs = [], []
    m = jnp.max(s, axis=1, keepdims=True)
    for t in range(TOPK):
        candf = jnp.where(s == m, colf, jnp.float32(3e38))
        amf = jnp.min(candf, axis=1, keepdims=True)
        vs.append(m)
        idfs.append(amf)
        if t < TOPK - 1:
            m = jnp.max(jnp.where(s < m, s, NEG), axis=1, keepdims=True)
    vs.append(jnp.full((B, 3), NEG, jnp.float32))
    idfs.append(jnp.full((B, 3), -base, jnp.float32))
    vals = jnp.concatenate(vs, axis=1).T
    idx = (jnp.concatenate(idfs, axis=1).T + base).astype(jnp.int32)
    return vals, idx


def _phase1_body(q_ref, keys_ref, vals_ref, idx_ref,
                 x_s, colf_s, s_a, s_b):
    k = pl.program_id(0)

    @pl.when(k == 0)
    def _():
        q = q_ref[...]
        mean = jnp.mean(q, axis=1, keepdims=True)
        var = jnp.mean((q - mean) ** 2, axis=1, keepdims=True)
        xn = (q - mean) / jnp.sqrt(var + 1e-5)
        xt = xn[:, :D]
        nrm = jnp.sqrt(jnp.sum(xt * xt, axis=1, keepdims=True))
        x_s[...] = xt / jnp.maximum(nrm, 1e-12)
        colf_s[...] = jax.lax.broadcasted_iota(
            jnp.int32, (B, KB), 1).astype(jnp.float32)

    x = x_s[...]
    kb = keys_ref[...]
    colf = colf_s[...]
    kpf = (k - 1).astype(jnp.float32)
    for j, buf in enumerate((s_a, s_b)):
        base = (kpf * UN + j) * jnp.float32(KB)
        vals, idx = _select_top5(buf[...], colf, base)
        vals_ref[j] = vals
        idx_ref[j] = idx
        buf[...] = jax.lax.dot_general(
            x, kb[j * KB:(j + 1) * KB, :], (((1,), (1,)), ((), ())),
            preferred_element_type=jnp.float32)


_phase1 = pl.pallas_call(
    _phase1_body,
    grid=(NSTEP,),
    in_specs=[
        pl.BlockSpec((B, QD), lambda k: (0, 0)),
        pl.BlockSpec((UN * KB, D),
                     lambda k: (jnp.minimum(k, NK // UN - 1), 0)),
    ],
    out_specs=[
        pl.BlockSpec((UN, 8, B), lambda k: (jnp.maximum(k - 1, 0), 0, 0)),
        pl.BlockSpec((UN, 8, B), lambda k: (jnp.maximum(k - 1, 0), 0, 0)),
    ],
    out_shape=[
        jax.ShapeDtypeStruct((NK, 8, B), jnp.float32),
        jax.ShapeDtypeStruct((NK, 8, B), jnp.int32),
    ],
    scratch_shapes=[pltpu.VMEM((B, D), jnp.float32),
                    pltpu.VMEM((B, KB), jnp.float32),
                    pltpu.VMEM((B, KB), jnp.float32),
                    pltpu.VMEM((B, KB), jnp.float32)],
)


def _smax(v):
    return jax.lax.sort(v)[15]


def _smin(v):
    return jax.lax.sort(v)[0]


def _sc_merge_body(cv_hbm, ci_hbm, ov_hbm, oi_hbm, v_s, i_s, obv, obi):
    c = jax.lax.axis_index("c")
    s = jax.lax.axis_index("s")
    wid = s * 2 + c
    base = wid * QPW
    pltpu.sync_copy(cv_hbm.at[pl.ds(base, QPW)], v_s)
    pltpu.sync_copy(ci_hbm.at[pl.ds(base, QPW)], i_s)
    lanes = jax.lax.iota(jnp.int32, 16)

    def per_query(j, carry):
        vacc = jnp.zeros((16,), jnp.float32)
        iacc = jnp.zeros((16,), jnp.int32)
        for t in range(TOPK):
            def per_chunk(cidx, bc):
                best, bpos = bc
                v = v_s[j, pl.ds(cidx * 16, 16)]
                m = v > best
                pos = cidx * 16 + lanes
                return jnp.where(m, v, best), jnp.where(m, pos, bpos)

            best, bpos = jax.lax.fori_loop(
                0, NCH, per_chunk,
                (jnp.full((16,), NEG, jnp.float32),
                 jnp.full((16,), 2 ** 30, jnp.int32)))
            gmax = _smax(best)
            cand = jnp.where(best == gmax, bpos, jnp.int32(2 ** 30))
            pmin = _smin(cand)
            ch = pmin // 16
            ln = pmin - ch * 16
            vv = v_s[j, pl.ds(ch * 16, 16)]
            iv = i_s[j, pl.ds(ch * 16, 16)]
            idxv = _smin(jnp.where(lanes == ln, iv, jnp.int32(2 ** 30)))
            v_s[j, pl.ds(ch * 16, 16)] = jnp.where(lanes == ln, NEG, vv)
            vacc = jnp.where(lanes == t, gmax, vacc)
            iacc = jnp.where(lanes == t, idxv, iacc)
        obv[j, pl.ds(0, 16)] = vacc
        obi[j, pl.ds(0, 16)] = iacc
        return carry

    jax.lax.fori_loop(0, QPW, per_query, 0)
    pltpu.sync_copy(obv, ov_hbm.at[pl.ds(base, QPW)])
    pltpu.sync_copy(obi, oi_hbm.at[pl.ds(base, QPW)])


@functools.cache
def _build_sc_merge():
    return functools.partial(
        pl.kernel,
        out_type=[
            jax.ShapeDtypeStruct((B, 16), jnp.float32),
            jax.ShapeDtypeStruct((B, 16), jnp.int32),
        ],
        mesh=plsc.VectorSubcoreMesh(core_axis_name="c", subcore_axis_name="s"),
        compiler_params=pltpu.CompilerParams(needs_layout_passes=False),
        scratch_types=[
            pltpu.VMEM((QPW, WPAD), jnp.float32),
            pltpu.VMEM((QPW, WPAD), jnp.int32),
            pltpu.VMEM((QPW, 16), jnp.float32),
            pltpu.VMEM((QPW, 16), jnp.int32),
        ],
    )(_sc_merge_body)


def kernel(queries, keys):
    cv, ci = _phase1(queries, keys)
    cv = jnp.transpose(cv, (2, 0, 1)).reshape(B, W)
    ci = jnp.transpose(ci, (2, 0, 1)).reshape(B, W)
    ov, oi = _build_sc_merge()(cv, ci)
    return ov[:, :TOPK], oi[:, :TOPK]


# speedup vs baseline: 2.8531x; 1.0006x over previous
import functools

import jax
import jax.numpy as jnp
from jax.experimental import pallas as pl
from jax.experimental.pallas import tpu as pltpu
from jax.experimental.pallas import tpu_sc as plsc

B = 1024
QD = 1024
D = 768
KTOT = 100000
KB = 1000
NK = 100
UN = 2
NSTEP = NK // UN + 1
TOPK = 5
NEG = -1e30

W = NK * 8
WPAD = W
NCH = WPAD // 16

NW = 32
QPW = B // NW


def _select_top5(s, colf, base):
    vs, idfs = [], []
    m = jnp.max(s, axis=1, keepdims=True)
    for t in range(TOPK):
        candf = jnp.where(s == m, colf, jnp.float32(3e38))
        amf = jnp.min(candf, axis=1, keepdims=True)
        vs.append(m)
        idfs.append(amf)
        if t < TOPK - 1:
            m = jnp.max(jnp.where(s < m, s, NEG), axis=1, keepdims=True)
    vs.append(jnp.full((B, 3), NEG, jnp.float32))
    idfs.append(jnp.full((B, 3), -base, jnp.float32))
    vals = jnp.concatenate(vs, axis=1).T
    idx = (jnp.concatenate(idfs, axis=1).T + base).astype(jnp.int32)
    return vals, idx


def _phase1_body(x_ref, keys_ref, vals_ref, idx_ref, colf_s, s_a, s_b):
    k = pl.program_id(0)

    @pl.when(k == 0)
    def _():
        colf_s[...] = jax.lax.broadcasted_iota(
            jnp.int32, (B, KB), 1).astype(jnp.float32)

    x = x_ref[...]
    kb = keys_ref[...]
    colf = colf_s[...]
    kpf = (k - 1).astype(jnp.float32)
    for j, buf in enumerate((s_a, s_b)):
        base = (kpf * UN + j) * jnp.float32(KB)
        vals, idx = _select_top5(buf[...], colf, base)
        vals_ref[j] = vals
        idx_ref[j] = idx
        buf[...] = jax.lax.dot_general(
            x, kb[j * KB:(j + 1) * KB, :], (((1,), (1,)), ((), ())),
            preferred_element_type=jnp.float32)


_phase1 = pl.pallas_call(
    _phase1_body,
    grid=(NSTEP,),
    in_specs=[
        pl.BlockSpec((B, D), lambda k: (0, 0)),
        pl.BlockSpec((UN * KB, D),
                     lambda k: (jnp.minimum(k, NK // UN - 1), 0)),
    ],
    out_specs=[
        pl.BlockSpec((UN, 8, B), lambda k: (jnp.maximum(k - 1, 0), 0, 0)),
        pl.BlockSpec((UN, 8, B), lambda k: (jnp.maximum(k - 1, 0), 0, 0)),
    ],
    out_shape=[
        jax.ShapeDtypeStruct((NK, 8, B), jnp.float32),
        jax.ShapeDtypeStruct((NK, 8, B), jnp.int32),
    ],
    scratch_shapes=[pltpu.VMEM((B, KB), jnp.float32),
                    pltpu.VMEM((B, KB), jnp.float32),
                    pltpu.VMEM((B, KB), jnp.float32)],
)


def _smax(v):
    return jax.lax.sort(v)[15]


def _smin(v):
    return jax.lax.sort(v)[0]


def _sc_merge_body(cv_hbm, ci_hbm, ov_hbm, oi_hbm, v_s, i_s, obv, obi):
    c = jax.lax.axis_index("c")
    s = jax.lax.axis_index("s")
    wid = s * 2 + c
    base = wid * QPW
    pltpu.sync_copy(cv_hbm.at[pl.ds(base, QPW)], v_s)
    pltpu.sync_copy(ci_hbm.at[pl.ds(base, QPW)], i_s)
    lanes = jax.lax.iota(jnp.int32, 16)

    def per_query(j, carry):
        vacc = jnp.zeros((16,), jnp.float32)
        iacc = jnp.zeros((16,), jnp.int32)
        for t in range(TOPK):
            def per_chunk(cidx, bc):
                best, bpos = bc
                v = v_s[j, pl.ds(cidx * 16, 16)]
                m = v > best
                pos = cidx * 16 + lanes
                return jnp.where(m, v, best), jnp.where(m, pos, bpos)

            best, bpos = jax.lax.fori_loop(
                0, NCH, per_chunk,
                (jnp.full((16,), NEG, jnp.float32),
                 jnp.full((16,), 2 ** 30, jnp.int32)))
            gmax = _smax(best)
            cand = jnp.where(best == gmax, bpos, jnp.int32(2 ** 30))
            pmin = _smin(cand)
            ch = pmin // 16
            ln = pmin - ch * 16
            vv = v_s[j, pl.ds(ch * 16, 16)]
            iv = i_s[j, pl.ds(ch * 16, 16)]
            idxv = _smin(jnp.where(lanes == ln, iv, jnp.int32(2 ** 30)))
            v_s[j, pl.ds(ch * 16, 16)] = jnp.where(lanes == ln, NEG, vv)
            vacc = jnp.where(lanes == t, gmax, vacc)
            iacc = jnp.where(lanes == t, idxv, iacc)
        obv[j, pl.ds(0, 16)] = vacc
        obi[j, pl.ds(0, 16)] = iacc
        return carry

    jax.lax.fori_loop(0, QPW, per_query, 0)
    pltpu.sync_copy(obv, ov_hbm.at[pl.ds(base, QPW)])
    pltpu.sync_copy(obi, oi_hbm.at[pl.ds(base, QPW)])


@functools.cache
def _build_sc_merge():
    return functools.partial(
        pl.kernel,
        out_type=[
            jax.ShapeDtypeStruct((B, 16), jnp.float32),
            jax.ShapeDtypeStruct((B, 16), jnp.int32),
        ],
        mesh=plsc.VectorSubcoreMesh(core_axis_name="c", subcore_axis_name="s"),
        compiler_params=pltpu.CompilerParams(needs_layout_passes=False),
        scratch_types=[
            pltpu.VMEM((QPW, WPAD), jnp.float32),
            pltpu.VMEM((QPW, WPAD), jnp.int32),
            pltpu.VMEM((QPW, 16), jnp.float32),
            pltpu.VMEM((QPW, 16), jnp.int32),
        ],
    )(_sc_merge_body)


def kernel(queries, keys):
    mean = jnp.mean(queries, axis=-1, keepdims=True)
    var = jnp.mean((queries - mean) ** 2, axis=-1, keepdims=True)
    x = (queries - mean) / jnp.sqrt(var + 1e-5)
    x = x[:, :D]
    x = x / jnp.clip(jnp.linalg.norm(x, axis=1, keepdims=True), 1e-12)
    cv, ci = _phase1(x, keys)
    cv = jnp.transpose(cv, (2, 0, 1)).reshape(B, W)
    ci = jnp.transpose(ci, (2, 0, 1)).reshape(B, W)
    ov, oi = _build_sc_merge()(cv, ci)
    return ov[:, :TOPK], oi[:, :TOPK]
